# baseline (speedup 1.0000x reference)
import functools
import jax
import jax.numpy as jnp
from jax import lax
from jax.experimental import pallas as pl
from jax.experimental.pallas import tpu as pltpu
from jax.experimental.pallas import tpu_sc as plsc

NUM_DEVICE_TYPES = 100000
EMBED_DIM = 64
BATCH = 4096
HIST_LEN = 50
SCALE = 8.0

NC = 2
NS = 16
NW = NC * NS
NROUND = EMBED_DIM // NW
UNROLL = 8


@functools.partial(
    pl.kernel,
    mesh=plsc.VectorSubcoreMesh(core_axis_name="c", subcore_axis_name="s"),
    out_type=jax.ShapeDtypeStruct((HIST_LEN, EMBED_DIM, BATCH // 16, 16),
                                   jnp.float32),
    scratch_types=[
        pltpu.VMEM_SHARED((HIST_LEN, BATCH // 16, 16), jnp.int32),
        pltpu.VMEM((NUM_DEVICE_TYPES,), jnp.float32),
        pltpu.VMEM((2, BATCH // 16, 16), jnp.int32),
        pltpu.VMEM((2, BATCH // 16, 16), jnp.float32),
        pltpu.SemaphoreType.DMA,
        pltpu.SemaphoreType.DMA((2,)),
        pltpu.SemaphoreType.DMA((2,)),
    ],
    compiler_params=pltpu.CompilerParams(use_tc_tiling_on_sc=False,
                                         needs_layout_passes=False),
)
def _embed_gather(tableT_hbm, idxT_hbm, out_hbm, idx_sh, trow_v, icol_v,
                  ocol_v, tsem, isem, osem):
    cid = lax.axis_index("c")
    sid = lax.axis_index("s")
    wid = sid * NC + cid

    @pl.when(sid == 0)
    def _():
        pltpu.sync_copy(idxT_hbm, idx_sh)

    plsc.subcore_barrier()

    def icol_copy(h, ib):
        return pltpu.make_async_copy(idx_sh.at[h], icol_v.at[ib], isem.at[ib])

    def trow_copy(d):
        return pltpu.make_async_copy(tableT_hbm.at[d], trow_v, tsem)

    def ocol_copy(h, d, ob):
        return pltpu.make_async_copy(ocol_v.at[ob], out_hbm.at[h, d],
                                     osem.at[ob])

    for rnd in range(NROUND):
        d = rnd * NW + wid
        trow_copy(d).start()
        icol_copy(0, 0).start()
        trow_copy(d).wait()

        def col_body(h, carry):
            ib = h % 2

            @pl.when(h + 1 < HIST_LEN)
            def _():
                icol_copy(h + 1, (h + 1) % 2).start()

            icol_copy(h, ib).wait()

            @pl.when(h >= 2)
            def _():
                ocol_copy(h - 2, d, ib).wait()

            icol = icol_v.at[ib]
            ocol = ocol_v.at[ib]

            def gloop(q, c2):
                for u in range(UNROLL):
                    r = q * UNROLL + u
                    vals = plsc.load_gather(trow_v, [icol[r, :]])
                    ocol[r, :] = vals * SCALE
                return c2

            lax.fori_loop(0, BATCH // (16 * UNROLL), gloop, 0)
            ocol_copy(h, d, ib).start()
            return carry

        lax.fori_loop(0, HIST_LEN, col_body, 0)

        for h in range(HIST_LEN - 2, HIST_LEN):
            ocol_copy(h, d, h % 2).wait()


def kernel(x, table):
    idxT = x.astype(jnp.int32).T.reshape(HIST_LEN, BATCH // 16, 16)
    tableT = table.T
    out = _embed_gather(tableT, idxT)
    return out.reshape(HIST_LEN, EMBED_DIM, BATCH).transpose(2, 0, 1)


# speedup vs baseline: 5.2015x; 5.2015x over previous
import functools
import jax
import jax.numpy as jnp
from jax import lax
from jax.experimental import pallas as pl
from jax.experimental.pallas import tpu as pltpu
from jax.experimental.pallas import tpu_sc as plsc

NUM_DEVICE_TYPES = 100000
EMBED_DIM = 64
BATCH = 4096
HIST_LEN = 50
SCALE = 8.0

NC = 2
NS = 16
NW = NC * NS
NROUND = EMBED_DIM // NW
UNROLL = 8


@functools.partial(
    pl.kernel,
    mesh=plsc.VectorSubcoreMesh(core_axis_name="c", subcore_axis_name="s"),
    out_type=jax.ShapeDtypeStruct((HIST_LEN, EMBED_DIM, BATCH), jnp.float32),
    scratch_types=[
        pltpu.VMEM_SHARED((HIST_LEN, BATCH), jnp.int32),
        pltpu.VMEM((NUM_DEVICE_TYPES,), jnp.float32),
        pltpu.VMEM((2, BATCH), jnp.int32),
        pltpu.VMEM((2, BATCH), jnp.float32),
        pltpu.SemaphoreType.DMA,
        pltpu.SemaphoreType.DMA((2,)),
        pltpu.SemaphoreType.DMA((2,)),
    ],
    compiler_params=pltpu.CompilerParams(use_tc_tiling_on_sc=False,
                                         needs_layout_passes=False),
)
def _embed_gather(tableT_hbm, idxT_hbm, out_hbm, idx_sh, trow_v, icol_v,
                  ocol_v, tsem, isem, osem):
    cid = lax.axis_index("c")
    sid = lax.axis_index("s")
    wid = sid * NC + cid

    def icol_copy(h, ib):
        return pltpu.make_async_copy(idx_sh.at[h], icol_v.at[ib], isem.at[ib])

    def trow_copy(d):
        return pltpu.make_async_copy(tableT_hbm.at[d], trow_v, tsem)

    def ocol_copy(h, d, ob):
        return pltpu.make_async_copy(ocol_v.at[ob], out_hbm.at[h, d],
                                     osem.at[ob])

    trow_copy(wid).start()

    @pl.when(sid == 0)
    def _():
        pltpu.sync_copy(idxT_hbm, idx_sh)

    plsc.subcore_barrier()

    for rnd in range(NROUND):
        d = rnd * NW + wid

        @pl.when(rnd > 0)
        def _():
            trow_copy(d).start()

        icol_copy(0, 0).start()
        trow_copy(d).wait()

        def col_body(h, carry):
            ib = h % 2

            @pl.when(h + 1 < HIST_LEN)
            def _():
                icol_copy(h + 1, (h + 1) % 2).start()

            icol_copy(h, ib).wait()

            @pl.when(h >= 2)
            def _():
                ocol_copy(h - 2, d, ib).wait()

            icol = icol_v.at[ib]
            ocol = ocol_v.at[ib]

            GB = 32
            for u0 in range(0, BATCH // 16, GB):
                vals = [
                    plsc.load_gather(trow_v, [icol[pl.ds(u * 16, 16)]])
                    for u in range(u0, u0 + GB)
                ]
                for i, u in enumerate(range(u0, u0 + GB)):
                    ocol[pl.ds(u * 16, 16)] = vals[i] * SCALE
            ocol_copy(h, d, ib).start()
            return carry

        lax.fori_loop(0, HIST_LEN, col_body, 0)

        for h in range(HIST_LEN - 2, HIST_LEN):
            ocol_copy(h, d, h % 2).wait()


def kernel(x, table):
    idxT = x.astype(jnp.int32).T
    tableT = table.T
    out = _embed_gather(tableT, idxT)
    return out.transpose(2, 0, 1)
